# baseline (speedup 1.0000x reference)
import functools

import jax
import jax.numpy as jnp
from jax import lax
from jax.experimental import pallas as pl
from jax.experimental.pallas import tpu as pltpu
from jax.experimental.pallas import tpu_sc as plsc


def _build_emb_kernel(B, H, D, HC, num_cores, num_subcores):
    NW = num_cores * num_subcores
    BL = B // NW
    n_steps = H // HC
    mesh = plsc.VectorSubcoreMesh(core_axis_name="c", subcore_axis_name="s")

    @functools.partial(
        pl.kernel,
        mesh=mesh,
        out_type=jax.ShapeDtypeStruct((H, D // 8, NW, 8, 128), jnp.float32),
        compiler_params=pltpu.CompilerParams(
            use_tc_tiling_on_sc=False, needs_layout_passes=False),
        scratch_types=[
            pltpu.VMEM((H, BL), jnp.int32),
            pltpu.VMEM((HC * BL, D), jnp.float32),
            pltpu.VMEM((HC * BL, D), jnp.float32),
            pltpu.VMEM((HC, D // 8, 8, BL), jnp.float32),
            pltpu.SemaphoreType.DMA,
            pltpu.SemaphoreType.DMA,
            pltpu.SemaphoreType.DMA,
        ],
    )
    def emb_kernel(idx_hbm, table_hbm, out_hbm, idx_v, gbuf0, gbuf1, tbuf,
                   gsem0, gsem1, ssem):
        wid = lax.axis_index("s") * num_cores + lax.axis_index("c")
        pltpu.sync_copy(idx_hbm.at[:, pl.ds(wid * BL, BL)], idx_v)

        gbufs = (gbuf0, gbuf1)
        gsems = (gsem0, gsem1)
        iota = lax.iota(jnp.int32, 16)
        biota = [iota + (k * 16) for k in range(BL // 16)]

        def gather(s, p):
            h0 = s * HC
            return [
                pltpu.async_copy(
                    table_hbm.at[idx_v.at[h0 + hi]],
                    gbufs[p].at[pl.ds(hi * BL, BL)], gsems[p])
                for hi in range(HC)
            ]

        def transpose(p):
            g = gbufs[p]

            def body(d, carry):
                dt = d // 8
                dr = d - dt * 8
                cidx = jnp.zeros((16,), jnp.int32) + d
                for hi in range(HC):
                    for k in range(BL // 16):
                        ridx = biota[k] + (hi * BL)
                        v = plsc.load_gather(g, [ridx, cidx])
                        tbuf[hi, dt, dr, pl.ds(k * 16, 16)] = v
                return carry

            lax.fori_loop(0, D, body, 0)

        sh = None
        gh = gather(0, 0)
        for s in range(n_steps):
            p = s % 2
            for h in gh:
                h.wait()
            if s + 1 < n_steps:
                gh = gather(s + 1, (s + 1) % 2)
            if sh is not None:
                sh.wait()
            transpose(p)
            sh = pltpu.async_copy(
                tbuf, out_hbm.at[pl.ds(s * HC, HC), :, wid], ssem)
        sh.wait()

    return emb_kernel


def kernel(x, table):
    B, H = x.shape
    V, D = table.shape

    info = plsc.get_sparse_core_info()
    HC = 5

    xt = jnp.transpose(x).astype(jnp.int32)
    out5 = _build_emb_kernel(B, H, D, HC, info.num_cores, info.num_subcores)(
        xt, table)
    return out5.transpose(2, 4, 0, 1, 3).reshape(B, H, D)
